# baseline (speedup 1.0000x reference)
import jax
import jax.numpy as jnp
import numpy as np
from jax import lax
from jax.experimental import pallas as pl
from jax.experimental.pallas import tpu as pltpu
from jax.experimental.pallas import tpu_sc as plsc

BATCH = 1024
CONTEXT = 20
EMB_DIM = 64
AUG = EMB_DIM + 1
VOCAB = 100000

VT = 2176
NV = -(-VOCAB // VT)
VPAD = NV * VT
VT2 = 2176
NV2 = VPAD // VT2
NEG = -1e30

NUM_WORKERS = 32
N_IDX = BATCH * CONTEXT
B_PER_W = N_IDX // NUM_WORKERS
ROWS_PER_W = BATCH // NUM_WORKERS
LCHUNK = 16


def _sc_gather_sum_body(table_hbm, idx_hbm, out_hbm, idx_v, rows_v, acc_v, sem):
    wid = lax.axis_index("s") * 2 + lax.axis_index("c")
    col0 = wid * ROWS_PER_W
    for c in range(CONTEXT):
        pltpu.sync_copy(
            idx_hbm.at[pl.ds(c * BATCH + col0, ROWS_PER_W)],
            idx_v.at[pl.ds(c * ROWS_PER_W, ROWS_PER_W)],
        )
    pltpu.async_copy(table_hbm.at[idx_v], rows_v, sem).wait()

    @pl.loop(0, ROWS_PER_W)
    def _(r):
        for k in range(EMB_DIM // LCHUNK):
            sl = pl.ds(k * LCHUNK, LCHUNK)
            acc = rows_v[r, sl]
            for c in range(1, CONTEXT):
                acc = acc + rows_v[c * ROWS_PER_W + r, sl]
            acc_v[r, sl] = acc

    pltpu.sync_copy(acc_v, out_hbm.at[pl.ds(col0, ROWS_PER_W)])


def _sc_gather_sum(table, idx):
    mesh = plsc.VectorSubcoreMesh(core_axis_name="c", subcore_axis_name="s")
    k = pl.kernel(
        _sc_gather_sum_body,
        out_type=jax.ShapeDtypeStruct((BATCH, EMB_DIM), jnp.float32),
        mesh=mesh,
        compiler_params=pltpu.CompilerParams(use_tc_tiling_on_sc=False),
        scratch_types=[
            pltpu.VMEM((B_PER_W,), jnp.int32),
            pltpu.VMEM((B_PER_W, EMB_DIM), jnp.float32),
            pltpu.VMEM((ROWS_PER_W, EMB_DIM), jnp.float32),
            pltpu.SemaphoreType.DMA,
        ],
    )
    return k(table, idx)


def _stats_body(x_ref, w_ref, lse_ref, m_ref):
    j = pl.program_id(0)

    @pl.when(j == 0)
    def _init():
        m_ref[...] = jnp.full((1, BATCH), -jnp.inf, jnp.float32)
        lse_ref[...] = jnp.zeros((1, BATCH), jnp.float32)

    lt = lax.dot_general(
        w_ref[...], x_ref[...].astype(jnp.bfloat16), (((1,), (0,)), ((), ())),
        preferred_element_type=jnp.float32,
    )
    tmax = jnp.max(lt, axis=0, keepdims=True)
    m_old = m_ref[...]
    m_new = jnp.maximum(m_old, tmax)
    s = lse_ref[...] * jnp.exp(m_old - m_new)
    s = s + jnp.sum(jnp.exp(lt - m_new), axis=0, keepdims=True)
    lse_ref[...] = s
    m_ref[...] = m_new

    @pl.when(j == NV - 1)
    def _fin():
        lse_ref[...] = m_ref[...] + jnp.log(lse_ref[...])


def _stats(x_aug_t, w_aug):
    return pl.pallas_call(
        _stats_body,
        grid=(NV,),
        in_specs=[
            pl.BlockSpec((AUG, BATCH), lambda j: (0, 0)),
            pl.BlockSpec((VT, AUG), lambda j: (j, 0)),
        ],
        out_specs=[
            pl.BlockSpec((1, BATCH), lambda j: (0, 0)),
            pl.BlockSpec((1, BATCH), lambda j: (0, 0)),
        ],
        out_shape=[
            jax.ShapeDtypeStruct((1, BATCH), jnp.float32),
            jax.ShapeDtypeStruct((1, BATCH), jnp.float32),
        ],
    )(x_aug_t, w_aug)


def _out_body(x_ref, w_ref, lse_ref, o_ref):
    lt = lax.dot_general(
        w_ref[...], x_ref[...].astype(jnp.bfloat16), (((1,), (0,)), ((), ())),
        preferred_element_type=jnp.float32,
    )
    o_ref[...] = lt - lse_ref[...]


def _final(x_aug_t, w_aug, lse):
    return pl.pallas_call(
        _out_body,
        grid=(NV2,),
        in_specs=[
            pl.BlockSpec((AUG, BATCH), lambda j: (0, 0)),
            pl.BlockSpec((VT2, AUG), lambda j: (j, 0)),
            pl.BlockSpec((1, BATCH), lambda j: (0, 0)),
        ],
        out_specs=pl.BlockSpec((VT2, BATCH), lambda j: (j, 0)),
        out_shape=jax.ShapeDtypeStruct((VOCAB, BATCH), jnp.float32),
    )(x_aug_t, w_aug, lse)


def kernel(inputs, emb_table, W, b):
    idx = inputs.astype(jnp.int32).T.reshape(-1)
    xsum = _sc_gather_sum(emb_table, idx)

    x_aug_t = jnp.concatenate(
        [xsum, jnp.ones((BATCH, 1), jnp.float32)], axis=1).T
    tail = np.zeros((VPAD - VOCAB, AUG), np.float32)
    tail[:, EMB_DIM] = NEG
    w_aug = jnp.concatenate(
        [jnp.concatenate([W, b[:, None]], axis=1), jnp.asarray(tail)],
        axis=0).astype(jnp.bfloat16)

    lse, _ = _stats(x_aug_t, w_aug)
    out_t = _final(x_aug_t, w_aug, lse)
    return out_t.T


# speedup vs baseline: 1.0740x; 1.0740x over previous
import jax
import jax.numpy as jnp
import numpy as np
from jax import lax
from jax.experimental import pallas as pl
from jax.experimental.pallas import tpu as pltpu
from jax.experimental.pallas import tpu_sc as plsc

BATCH = 1024
CONTEXT = 20
EMB_DIM = 64
AUG = EMB_DIM + 1
VOCAB = 100000

VT = 2176
NV = -(-VOCAB // VT)
VPAD = NV * VT
VT2 = 2176
NV2 = VPAD // VT2
NEG = -1e30

NUM_WORKERS = 32
N_IDX = BATCH * CONTEXT
B_PER_W = N_IDX // NUM_WORKERS
ROWS_PER_W = BATCH // NUM_WORKERS
LCHUNK = 16


def _sc_gather_sum_body(table_hbm, idx_hbm, out_hbm, idx_v, rows_v, acc_v, sem):
    wid = lax.axis_index("s") * 2 + lax.axis_index("c")
    col0 = wid * ROWS_PER_W
    for c in range(CONTEXT):
        pltpu.sync_copy(
            idx_hbm.at[pl.ds(c * BATCH + col0, ROWS_PER_W)],
            idx_v.at[pl.ds(c * ROWS_PER_W, ROWS_PER_W)],
        )
    pltpu.async_copy(table_hbm.at[idx_v], rows_v, sem).wait()

    @pl.loop(0, ROWS_PER_W)
    def _(r):
        for k in range(EMB_DIM // LCHUNK):
            sl = pl.ds(k * LCHUNK, LCHUNK)
            acc = rows_v[r, sl]
            for c in range(1, CONTEXT):
                acc = acc + rows_v[c * ROWS_PER_W + r, sl]
            acc_v[r, sl] = acc

    pltpu.sync_copy(acc_v, out_hbm.at[pl.ds(col0, ROWS_PER_W)])


def _sc_gather_sum(table, idx):
    mesh = plsc.VectorSubcoreMesh(core_axis_name="c", subcore_axis_name="s")
    k = pl.kernel(
        _sc_gather_sum_body,
        out_type=jax.ShapeDtypeStruct((BATCH, EMB_DIM), jnp.float32),
        mesh=mesh,
        compiler_params=pltpu.CompilerParams(use_tc_tiling_on_sc=False),
        scratch_types=[
            pltpu.VMEM((B_PER_W,), jnp.int32),
            pltpu.VMEM((B_PER_W, EMB_DIM), jnp.float32),
            pltpu.VMEM((ROWS_PER_W, EMB_DIM), jnp.float32),
            pltpu.SemaphoreType.DMA,
        ],
    )
    return k(table, idx)


def _stats_body(x_ref, w_ref, lse_ref, m_ref):
    j = pl.program_id(0)

    @pl.when(j == 0)
    def _init():
        m_ref[...] = jnp.full((1, BATCH), -jnp.inf, jnp.float32)
        lse_ref[...] = jnp.zeros((1, BATCH), jnp.float32)

    lt = lax.dot_general(
        w_ref[...], x_ref[...].astype(jnp.bfloat16), (((1,), (0,)), ((), ())),
        preferred_element_type=jnp.float32,
    )
    tmax = jnp.max(lt, axis=0, keepdims=True)
    m_old = m_ref[...]
    m_new = jnp.maximum(m_old, tmax)
    s = lse_ref[...] * jnp.exp(m_old - m_new)
    s = s + jnp.sum(jnp.exp(lt - m_new), axis=0, keepdims=True)
    lse_ref[...] = s
    m_ref[...] = m_new

    @pl.when(j == NV - 1)
    def _fin():
        lse_ref[...] = m_ref[...] + jnp.log(lse_ref[...])


def _stats(x_aug_t, w_aug):
    return pl.pallas_call(
        _stats_body,
        grid=(NV,),
        in_specs=[
            pl.BlockSpec((AUG, BATCH), lambda j: (0, 0)),
            pl.BlockSpec((VT, AUG), lambda j: (j, 0)),
        ],
        out_specs=[
            pl.BlockSpec((1, BATCH), lambda j: (0, 0)),
            pl.BlockSpec((1, BATCH), lambda j: (0, 0)),
        ],
        out_shape=[
            jax.ShapeDtypeStruct((1, BATCH), jnp.float32),
            jax.ShapeDtypeStruct((1, BATCH), jnp.float32),
        ],
    )(x_aug_t, w_aug)


def _out_body(x_ref, w_ref, lse_ref, o_ref):
    lt = lax.dot_general(
        w_ref[...], x_ref[...].astype(jnp.bfloat16), (((1,), (0,)), ((), ())),
        preferred_element_type=jnp.float32,
    )
    o_ref[...] = lt - lse_ref[...]


def _final(x_aug_t, w_aug, lse):
    return pl.pallas_call(
        _out_body,
        grid=(NV2,),
        in_specs=[
            pl.BlockSpec((AUG, BATCH), lambda j: (0, 0)),
            pl.BlockSpec((VT2, AUG), lambda j: (j, 0)),
            pl.BlockSpec((1, BATCH), lambda j: (0, 0)),
        ],
        out_specs=pl.BlockSpec((VT2, BATCH), lambda j: (j, 0)),
        out_shape=jax.ShapeDtypeStruct((VOCAB, BATCH), jnp.float32),
    )(x_aug_t, w_aug, lse)


def kernel(inputs, emb_table, W, b):
    idx = inputs.astype(jnp.int32).T.reshape(-1)
    xsum = _sc_gather_sum(emb_table, idx)

    x_aug_t = jnp.concatenate(
        [xsum, jnp.ones((BATCH, 1), jnp.float32)], axis=1).T
    w_aug = jnp.concatenate(
        [jnp.pad(W, ((0, VPAD - VOCAB), (0, 0))),
         jnp.pad(b, (0, VPAD - VOCAB), constant_values=NEG)[:, None]],
        axis=1).astype(jnp.bfloat16)

    lse, _ = _stats(x_aug_t, w_aug)
    out_t = _final(x_aug_t, w_aug, lse)
    return out_t.T


# speedup vs baseline: 1.2826x; 1.1942x over previous
import jax
import jax.numpy as jnp
import numpy as np
from jax import lax
from jax.experimental import pallas as pl
from jax.experimental.pallas import tpu as pltpu
from jax.experimental.pallas import tpu_sc as plsc

BATCH = 1024
CONTEXT = 20
EMB_DIM = 64
AUG = EMB_DIM + 1
VOCAB = 100000

VT = 2176
NV = -(-VOCAB // VT)
VPAD = NV * VT
VT2 = 2176
NV2 = VPAD // VT2
NEG = -1e30

NUM_WORKERS = 32
N_IDX = BATCH * CONTEXT
B_PER_W = N_IDX // NUM_WORKERS
ROWS_PER_W = BATCH // NUM_WORKERS
LCHUNK = 16


def _sc_gather_sum_body(table_hbm, idx_hbm, out_hbm, idx_v, rows_v, acc_v, sem):
    wid = lax.axis_index("s") * 2 + lax.axis_index("c")
    col0 = wid * ROWS_PER_W
    for c in range(CONTEXT):
        pltpu.sync_copy(
            idx_hbm.at[pl.ds(c * BATCH + col0, ROWS_PER_W)],
            idx_v.at[pl.ds(c * ROWS_PER_W, ROWS_PER_W)],
        )
    pltpu.async_copy(table_hbm.at[idx_v], rows_v, sem).wait()

    @pl.loop(0, ROWS_PER_W)
    def _(r):
        for k in range(EMB_DIM // LCHUNK):
            sl = pl.ds(k * LCHUNK, LCHUNK)
            acc = rows_v[r, sl]
            for c in range(1, CONTEXT):
                acc = acc + rows_v[c * ROWS_PER_W + r, sl]
            acc_v[r, sl] = acc

    pltpu.sync_copy(acc_v, out_hbm.at[pl.ds(col0, ROWS_PER_W)])


def _sc_gather_sum(table, idx):
    mesh = plsc.VectorSubcoreMesh(core_axis_name="c", subcore_axis_name="s")
    k = pl.kernel(
        _sc_gather_sum_body,
        out_type=jax.ShapeDtypeStruct((BATCH, EMB_DIM), jnp.float32),
        mesh=mesh,
        compiler_params=pltpu.CompilerParams(use_tc_tiling_on_sc=False),
        scratch_types=[
            pltpu.VMEM((B_PER_W,), jnp.int32),
            pltpu.VMEM((B_PER_W, EMB_DIM), jnp.float32),
            pltpu.VMEM((ROWS_PER_W, EMB_DIM), jnp.float32),
            pltpu.SemaphoreType.DMA,
        ],
    )
    return k(table, idx)


def _stats_body(x_ref, w_ref, lse_ref, m_ref):
    j = pl.program_id(0)

    @pl.when(j == 0)
    def _init():
        m_ref[...] = jnp.full((1, BATCH), -jnp.inf, jnp.float32)
        lse_ref[...] = jnp.zeros((1, BATCH), jnp.float32)

    lt = lax.dot_general(
        w_ref[...], x_ref[...].astype(jnp.bfloat16), (((0,), (0,)), ((), ())),
        preferred_element_type=jnp.float32,
    )
    tmax = jnp.max(lt, axis=0, keepdims=True)
    m_old = m_ref[...]
    m_new = jnp.maximum(m_old, tmax)
    s = lse_ref[...] * jnp.exp(m_old - m_new)
    s = s + jnp.sum(jnp.exp(lt - m_new), axis=0, keepdims=True)
    lse_ref[...] = s
    m_ref[...] = m_new

    @pl.when(j == NV - 1)
    def _fin():
        lse_ref[...] = m_ref[...] + jnp.log(lse_ref[...])


def _stats(x_aug_t, w_aug):
    return pl.pallas_call(
        _stats_body,
        grid=(NV,),
        in_specs=[
            pl.BlockSpec((AUG, BATCH), lambda j: (0, 0)),
            pl.BlockSpec((AUG, VT), lambda j: (0, j)),
        ],
        out_specs=[
            pl.BlockSpec((1, BATCH), lambda j: (0, 0)),
            pl.BlockSpec((1, BATCH), lambda j: (0, 0)),
        ],
        out_shape=[
            jax.ShapeDtypeStruct((1, BATCH), jnp.float32),
            jax.ShapeDtypeStruct((1, BATCH), jnp.float32),
        ],
    )(x_aug_t, w_aug)


def _out_body(x_ref, w_ref, lse_ref, o_ref):
    lt = lax.dot_general(
        w_ref[...], x_ref[...].astype(jnp.bfloat16), (((0,), (0,)), ((), ())),
        preferred_element_type=jnp.float32,
    )
    o_ref[...] = lt - lse_ref[...]


def _final(x_aug_t, w_aug, lse):
    return pl.pallas_call(
        _out_body,
        grid=(NV2,),
        in_specs=[
            pl.BlockSpec((AUG, BATCH), lambda j: (0, 0)),
            pl.BlockSpec((AUG, VT2), lambda j: (0, j)),
            pl.BlockSpec((1, BATCH), lambda j: (0, 0)),
        ],
        out_specs=pl.BlockSpec((VT2, BATCH), lambda j: (j, 0)),
        out_shape=jax.ShapeDtypeStruct((VOCAB, BATCH), jnp.float32),
    )(x_aug_t, w_aug, lse)


def kernel(inputs, emb_table, W, b):
    idx = inputs.astype(jnp.int32).T.reshape(-1)
    xsum = _sc_gather_sum(emb_table, idx)

    x_aug_t = jnp.concatenate(
        [xsum, jnp.ones((BATCH, 1), jnp.float32)], axis=1).T
    w_aug = jnp.concatenate(
        [jnp.pad(W.T, ((0, 0), (0, VPAD - VOCAB))),
         jnp.pad(b, (0, VPAD - VOCAB), constant_values=NEG)[None, :]],
        axis=0).astype(jnp.bfloat16)

    lse, _ = _stats(x_aug_t, w_aug)
    out_t = _final(x_aug_t, w_aug, lse)
    return out_t.T


# speedup vs baseline: 1.2836x; 1.0007x over previous
import jax
import jax.numpy as jnp
import numpy as np
from jax import lax
from jax.experimental import pallas as pl
from jax.experimental.pallas import tpu as pltpu
from jax.experimental.pallas import tpu_sc as plsc

BATCH = 1024
CONTEXT = 20
EMB_DIM = 64
AUG = EMB_DIM + 1
VOCAB = 100000

VT = 2176
NV = -(-VOCAB // VT)
VPAD = NV * VT
VT2 = 2176
NV2 = VPAD // VT2
NEG = -1e30
LOG2E = 1.4426950408889634
LN2 = 0.6931471805599453

NUM_WORKERS = 32
N_IDX = BATCH * CONTEXT
B_PER_W = N_IDX // NUM_WORKERS
ROWS_PER_W = BATCH // NUM_WORKERS
LCHUNK = 16


def _sc_gather_sum_body(table_hbm, idx_hbm, out_hbm, idx_v, rows_v, acc_v, sem):
    wid = lax.axis_index("s") * 2 + lax.axis_index("c")
    col0 = wid * ROWS_PER_W
    for c in range(CONTEXT):
        pltpu.sync_copy(
            idx_hbm.at[pl.ds(c * BATCH + col0, ROWS_PER_W)],
            idx_v.at[pl.ds(c * ROWS_PER_W, ROWS_PER_W)],
        )
    pltpu.async_copy(table_hbm.at[idx_v], rows_v, sem).wait()

    @pl.loop(0, ROWS_PER_W)
    def _(r):
        for k in range(EMB_DIM // LCHUNK):
            sl = pl.ds(k * LCHUNK, LCHUNK)
            acc = rows_v[r, sl]
            for c in range(1, CONTEXT):
                acc = acc + rows_v[c * ROWS_PER_W + r, sl]
            acc_v[r, sl] = acc

    pltpu.sync_copy(acc_v, out_hbm.at[pl.ds(col0, ROWS_PER_W)])


def _sc_gather_sum(table, idx):
    mesh = plsc.VectorSubcoreMesh(core_axis_name="c", subcore_axis_name="s")
    k = pl.kernel(
        _sc_gather_sum_body,
        out_type=jax.ShapeDtypeStruct((BATCH, EMB_DIM), jnp.float32),
        mesh=mesh,
        compiler_params=pltpu.CompilerParams(use_tc_tiling_on_sc=False),
        scratch_types=[
            pltpu.VMEM((B_PER_W,), jnp.int32),
            pltpu.VMEM((B_PER_W, EMB_DIM), jnp.float32),
            pltpu.VMEM((ROWS_PER_W, EMB_DIM), jnp.float32),
            pltpu.SemaphoreType.DMA,
        ],
    )
    return k(table, idx)


def _stats_body(x_ref, w_ref, lse_ref, m_ref):
    j = pl.program_id(0)

    @pl.when(j == 0)
    def _init():
        m_ref[...] = jnp.full((1, BATCH), -jnp.inf, jnp.float32)
        lse_ref[...] = jnp.zeros((1, BATCH), jnp.float32)

    lt2 = lax.dot_general(
        w_ref[...], x_ref[...].astype(jnp.bfloat16), (((0,), (0,)), ((), ())),
        preferred_element_type=jnp.float32,
    )
    tmax = jnp.max(lt2, axis=0, keepdims=True)
    m_old = m_ref[...]
    m_new = jnp.maximum(m_old, tmax)
    e = jnp.exp2(lt2 - m_new)
    tsum = jnp.sum(e, axis=0, keepdims=True)
    s = lse_ref[...] * jnp.exp2(m_old - m_new) + tsum
    lse_ref[...] = s
    m_ref[...] = m_new

    @pl.when(j == NV - 1)
    def _fin():
        lse_ref[...] = (m_ref[...] + jnp.log2(lse_ref[...])) * LN2


def _stats(x_aug_t, w_aug):
    return pl.pallas_call(
        _stats_body,
        grid=(NV,),
        in_specs=[
            pl.BlockSpec((AUG, BATCH), lambda j: (0, 0)),
            pl.BlockSpec((AUG, VT), lambda j: (0, j)),
        ],
        out_specs=[
            pl.BlockSpec((1, BATCH), lambda j: (0, 0)),
            pl.BlockSpec((1, BATCH), lambda j: (0, 0)),
        ],
        out_shape=[
            jax.ShapeDtypeStruct((1, BATCH), jnp.float32),
            jax.ShapeDtypeStruct((1, BATCH), jnp.float32),
        ],
    )(x_aug_t, w_aug)


def _out_body(x_ref, w_ref, lse_ref, o_ref):
    lt = lax.dot_general(
        w_ref[...], x_ref[...].astype(jnp.bfloat16), (((0,), (0,)), ((), ())),
        preferred_element_type=jnp.float32,
    )
    o_ref[...] = lt * LN2 - lse_ref[...]


def _final(x_aug_t, w_aug, lse):
    return pl.pallas_call(
        _out_body,
        grid=(NV2,),
        in_specs=[
            pl.BlockSpec((AUG, BATCH), lambda j: (0, 0)),
            pl.BlockSpec((AUG, VT2), lambda j: (0, j)),
            pl.BlockSpec((1, BATCH), lambda j: (0, 0)),
        ],
        out_specs=pl.BlockSpec((VT2, BATCH), lambda j: (j, 0)),
        out_shape=jax.ShapeDtypeStruct((VOCAB, BATCH), jnp.float32),
    )(x_aug_t, w_aug, lse)


def kernel(inputs, emb_table, W, b):
    idx = inputs.astype(jnp.int32).T.reshape(-1)
    xsum = _sc_gather_sum(emb_table, idx)

    x_aug_t = jnp.concatenate(
        [xsum, jnp.ones((BATCH, 1), jnp.float32)], axis=1).T
    w_aug = (jnp.concatenate(
        [jnp.pad(W.T, ((0, 0), (0, VPAD - VOCAB))),
         jnp.pad(b, (0, VPAD - VOCAB), constant_values=NEG)[None, :]],
        axis=0) * LOG2E).astype(jnp.bfloat16)

    lse, _ = _stats(x_aug_t, w_aug)
    out_t = _final(x_aug_t, w_aug, lse)
    return out_t.T


# speedup vs baseline: 1.2856x; 1.0016x over previous
import jax
import jax.numpy as jnp
import numpy as np
from jax import lax
from jax.experimental import pallas as pl
from jax.experimental.pallas import tpu as pltpu
from jax.experimental.pallas import tpu_sc as plsc

BATCH = 1024
CONTEXT = 20
EMB_DIM = 64
AUG = EMB_DIM + 1
VOCAB = 100000

VT = 2176
NV = -(-VOCAB // VT)
VPAD = NV * VT
VT2 = 2176
NV2 = VPAD // VT2
NEG = -1e30
LOG2E = 1.4426950408889634
LN2 = 0.6931471805599453

NUM_WORKERS = 32
N_IDX = BATCH * CONTEXT
B_PER_W = N_IDX // NUM_WORKERS
ROWS_PER_W = BATCH // NUM_WORKERS
LCHUNK = 16


def _idx_body(in_ref, o_ref):
    o_ref[...] = in_ref[...].T.reshape(N_IDX)


def _idx_flat(inputs):
    return pl.pallas_call(
        _idx_body,
        in_specs=[pl.BlockSpec((BATCH, CONTEXT), lambda: (0, 0))],
        out_specs=pl.BlockSpec((N_IDX,), lambda: (0,)),
        out_shape=jax.ShapeDtypeStruct((N_IDX,), jnp.int32),
    )(inputs)


def _sc_gather_sum_body(table_hbm, idx_hbm, out_hbm, idx_v, rows_v, acc_v, sem):
    wid = lax.axis_index("s") * 2 + lax.axis_index("c")
    col0 = wid * ROWS_PER_W
    for c in range(CONTEXT):
        pltpu.sync_copy(
            idx_hbm.at[pl.ds(c * BATCH + col0, ROWS_PER_W)],
            idx_v.at[pl.ds(c * ROWS_PER_W, ROWS_PER_W)],
        )
    pltpu.async_copy(table_hbm.at[idx_v], rows_v, sem).wait()

    @pl.loop(0, ROWS_PER_W)
    def _(r):
        for k in range(EMB_DIM // LCHUNK):
            sl = pl.ds(k * LCHUNK, LCHUNK)
            acc = rows_v[r, sl]
            for c in range(1, CONTEXT):
                acc = acc + rows_v[c * ROWS_PER_W + r, sl]
            acc_v[r, sl] = acc

    pltpu.sync_copy(acc_v, out_hbm.at[pl.ds(col0, ROWS_PER_W)])


def _sc_gather_sum(table, idx):
    mesh = plsc.VectorSubcoreMesh(core_axis_name="c", subcore_axis_name="s")
    k = pl.kernel(
        _sc_gather_sum_body,
        out_type=jax.ShapeDtypeStruct((BATCH, EMB_DIM), jnp.float32),
        mesh=mesh,
        compiler_params=pltpu.CompilerParams(use_tc_tiling_on_sc=False),
        scratch_types=[
            pltpu.VMEM((B_PER_W,), jnp.int32),
            pltpu.VMEM((B_PER_W, EMB_DIM), jnp.float32),
            pltpu.VMEM((ROWS_PER_W, EMB_DIM), jnp.float32),
            pltpu.SemaphoreType.DMA,
        ],
    )
    return k(table, idx)


def _stats_body(x_ref, w_ref, lse_ref, m_ref):
    j = pl.program_id(0)

    @pl.when(j == 0)
    def _init():
        m_ref[...] = jnp.full((1, BATCH), -jnp.inf, jnp.float32)
        lse_ref[...] = jnp.zeros((1, BATCH), jnp.float32)

    lt2 = lax.dot_general(
        w_ref[...], x_ref[...].astype(jnp.bfloat16), (((0,), (0,)), ((), ())),
        preferred_element_type=jnp.float32,
    )
    tmax = jnp.max(lt2, axis=0, keepdims=True)
    m_old = m_ref[...]
    m_new = jnp.maximum(m_old, tmax)
    e = jnp.exp2(lt2 - m_new)
    tsum = jnp.sum(e, axis=0, keepdims=True)
    s = lse_ref[...] * jnp.exp2(m_old - m_new) + tsum
    lse_ref[...] = s
    m_ref[...] = m_new

    @pl.when(j == NV - 1)
    def _fin():
        lse_ref[...] = (m_ref[...] + jnp.log2(lse_ref[...])) * LN2


def _stats(x_aug_t, w_aug):
    return pl.pallas_call(
        _stats_body,
        grid=(NV,),
        in_specs=[
            pl.BlockSpec((AUG, BATCH), lambda j: (0, 0)),
            pl.BlockSpec((AUG, VT), lambda j: (0, j)),
        ],
        out_specs=[
            pl.BlockSpec((1, BATCH), lambda j: (0, 0)),
            pl.BlockSpec((1, BATCH), lambda j: (0, 0)),
        ],
        out_shape=[
            jax.ShapeDtypeStruct((1, BATCH), jnp.float32),
            jax.ShapeDtypeStruct((1, BATCH), jnp.float32),
        ],
    )(x_aug_t, w_aug)


def _out_body(x_ref, w_ref, lse_ref, o_ref):
    lt = lax.dot_general(
        w_ref[...], x_ref[...].astype(jnp.bfloat16), (((0,), (0,)), ((), ())),
        preferred_element_type=jnp.float32,
    )
    o_ref[...] = lt * LN2 - lse_ref[...]


def _final(x_aug_t, w_aug, lse):
    return pl.pallas_call(
        _out_body,
        grid=(NV2,),
        in_specs=[
            pl.BlockSpec((AUG, BATCH), lambda j: (0, 0)),
            pl.BlockSpec((AUG, VT2), lambda j: (0, j)),
            pl.BlockSpec((1, BATCH), lambda j: (0, 0)),
        ],
        out_specs=pl.BlockSpec((VT2, BATCH), lambda j: (j, 0)),
        out_shape=jax.ShapeDtypeStruct((VOCAB, BATCH), jnp.float32),
    )(x_aug_t, w_aug, lse)


def kernel(inputs, emb_table, W, b):
    idx = _idx_flat(inputs.astype(jnp.int32))
    xsum = _sc_gather_sum(emb_table, idx)

    x_aug_t = jnp.concatenate(
        [xsum, jnp.ones((BATCH, 1), jnp.float32)], axis=1).T
    w_aug = (jnp.concatenate(
        [jnp.pad(W.T, ((0, 0), (0, VPAD - VOCAB))),
         jnp.pad(b, (0, VPAD - VOCAB), constant_values=NEG)[None, :]],
        axis=0) * LOG2E).astype(jnp.bfloat16)

    lse, _ = _stats(x_aug_t, w_aug)
    out_t = _final(x_aug_t, w_aug, lse)
    return out_t.T


# speedup vs baseline: 1.3200x; 1.0267x over previous
import jax
import jax.numpy as jnp
import numpy as np
from jax import lax
from jax.experimental import pallas as pl
from jax.experimental.pallas import tpu as pltpu
from jax.experimental.pallas import tpu_sc as plsc

BATCH = 1024
CONTEXT = 20
EMB_DIM = 64
AUG = EMB_DIM + 1
VOCAB = 100000

VT = 2176
NV = -(-VOCAB // VT)
VPAD = NV * VT
VT2 = 2176
NV2 = VPAD // VT2
NEG = -1e30
LOG2E = 1.4426950408889634
LN2 = 0.6931471805599453

NUM_WORKERS = 32
N_IDX = BATCH * CONTEXT
B_PER_W = N_IDX // NUM_WORKERS
ROWS_PER_W = BATCH // NUM_WORKERS
LCHUNK = 16


def _idx_body(in_ref, o_ref):
    o_ref[...] = in_ref[...].T.reshape(N_IDX)


def _idx_flat(inputs):
    return pl.pallas_call(
        _idx_body,
        in_specs=[pl.BlockSpec((BATCH, CONTEXT), lambda: (0, 0))],
        out_specs=pl.BlockSpec((N_IDX,), lambda: (0,)),
        out_shape=jax.ShapeDtypeStruct((N_IDX,), jnp.int32),
    )(inputs)


TPT = 2048

def _tbl_body(in_ref, o_ref):
    o_ref[...] = jnp.pad(in_ref[...].T, ((0, 0), (0, EMB_DIM)))


def _tbl_repack(table_t):
    return pl.pallas_call(
        _tbl_body,
        grid=(-(-VOCAB // TPT),),
        in_specs=[pl.BlockSpec((EMB_DIM, TPT), lambda j: (0, j))],
        out_specs=pl.BlockSpec((TPT, 2 * EMB_DIM), lambda j: (j, 0)),
        out_shape=jax.ShapeDtypeStruct((VOCAB, 2 * EMB_DIM), jnp.float32),
    )(table_t)


def _sc_gather_sum_body(table_hbm, idx_hbm, out_hbm, idx_v, rows_v, acc_v, sem):
    wid = lax.axis_index("s") * 2 + lax.axis_index("c")
    col0 = wid * ROWS_PER_W
    for c in range(CONTEXT):
        pltpu.sync_copy(
            idx_hbm.at[pl.ds(c * BATCH + col0, ROWS_PER_W)],
            idx_v.at[pl.ds(c * ROWS_PER_W, ROWS_PER_W)],
        )
    pltpu.async_copy(table_hbm.at[idx_v], rows_v, sem).wait()

    @pl.loop(0, ROWS_PER_W)
    def _(r):
        for k in range(EMB_DIM // LCHUNK):
            sl = pl.ds(k * LCHUNK, LCHUNK)
            acc = rows_v[r, sl]
            for c in range(1, CONTEXT):
                acc = acc + rows_v[c * ROWS_PER_W + r, sl]
            acc_v[r, sl] = acc

    pltpu.sync_copy(acc_v, out_hbm.at[pl.ds(col0, ROWS_PER_W)])


def _sc_gather_sum(table128, idx):
    mesh = plsc.VectorSubcoreMesh(core_axis_name="c", subcore_axis_name="s")
    k = pl.kernel(
        _sc_gather_sum_body,
        out_type=jax.ShapeDtypeStruct((BATCH, 2 * EMB_DIM), jnp.float32),
        mesh=mesh,
        compiler_params=pltpu.CompilerParams(use_tc_tiling_on_sc=True),
        scratch_types=[
            pltpu.VMEM((B_PER_W,), jnp.int32),
            pltpu.VMEM((B_PER_W, 2 * EMB_DIM), jnp.float32),
            pltpu.VMEM((ROWS_PER_W, 2 * EMB_DIM), jnp.float32),
            pltpu.SemaphoreType.DMA,
        ],
    )
    return k(table128, idx)


def _stats_body(x_ref, w_ref, lse_ref, m_ref):
    j = pl.program_id(0)

    @pl.when(j == 0)
    def _init():
        m_ref[...] = jnp.full((1, BATCH), -jnp.inf, jnp.float32)
        lse_ref[...] = jnp.zeros((1, BATCH), jnp.float32)

    lt2 = lax.dot_general(
        w_ref[...], x_ref[...].astype(jnp.bfloat16), (((0,), (0,)), ((), ())),
        preferred_element_type=jnp.float32,
    )
    tmax = jnp.max(lt2, axis=0, keepdims=True)
    m_old = m_ref[...]
    m_new = jnp.maximum(m_old, tmax)
    e = jnp.exp2(lt2 - m_new)
    tsum = jnp.sum(e, axis=0, keepdims=True)
    s = lse_ref[...] * jnp.exp2(m_old - m_new) + tsum
    lse_ref[...] = s
    m_ref[...] = m_new

    @pl.when(j == NV - 1)
    def _fin():
        lse_ref[...] = (m_ref[...] + jnp.log2(lse_ref[...])) * LN2


def _stats(x_aug_t, w_aug):
    return pl.pallas_call(
        _stats_body,
        grid=(NV,),
        in_specs=[
            pl.BlockSpec((AUG, BATCH), lambda j: (0, 0)),
            pl.BlockSpec((AUG, VT), lambda j: (0, j)),
        ],
        out_specs=[
            pl.BlockSpec((1, BATCH), lambda j: (0, 0)),
            pl.BlockSpec((1, BATCH), lambda j: (0, 0)),
        ],
        out_shape=[
            jax.ShapeDtypeStruct((1, BATCH), jnp.float32),
            jax.ShapeDtypeStruct((1, BATCH), jnp.float32),
        ],
    )(x_aug_t, w_aug)


def _out_body(x_ref, w_ref, lse_ref, o_ref):
    lt = lax.dot_general(
        w_ref[...], x_ref[...].astype(jnp.bfloat16), (((0,), (0,)), ((), ())),
        preferred_element_type=jnp.float32,
    )
    o_ref[...] = lt * LN2 - lse_ref[...]


def _final(x_aug_t, w_aug, lse):
    return pl.pallas_call(
        _out_body,
        grid=(NV2,),
        in_specs=[
            pl.BlockSpec((AUG, BATCH), lambda j: (0, 0)),
            pl.BlockSpec((AUG, VT2), lambda j: (0, j)),
            pl.BlockSpec((1, BATCH), lambda j: (0, 0)),
        ],
        out_specs=pl.BlockSpec((VT2, BATCH), lambda j: (j, 0)),
        out_shape=jax.ShapeDtypeStruct((VOCAB, BATCH), jnp.float32),
    )(x_aug_t, w_aug, lse)


def kernel(inputs, emb_table, W, b):
    idx = _idx_flat(inputs.astype(jnp.int32))
    table128 = _tbl_repack(emb_table.T)
    xsum = _sc_gather_sum(table128, idx)

    x_aug_t = jnp.concatenate(
        [xsum[:, :EMB_DIM], jnp.ones((BATCH, 1), jnp.float32)], axis=1).T
    w_aug = (jnp.concatenate(
        [jnp.pad(W.T, ((0, 0), (0, VPAD - VOCAB))),
         jnp.pad(b, (0, VPAD - VOCAB), constant_values=NEG)[None, :]],
        axis=0) * LOG2E).astype(jnp.bfloat16)

    lse, _ = _stats(x_aug_t, w_aug)
    out_t = _final(x_aug_t, w_aug, lse)
    return out_t.T


# speedup vs baseline: 1.3561x; 1.0274x over previous
import jax
import jax.numpy as jnp
import numpy as np
from jax import lax
from jax.experimental import pallas as pl
from jax.experimental.pallas import tpu as pltpu
from jax.experimental.pallas import tpu_sc as plsc

BATCH = 1024
VPAD_ = 100096
CONTEXT = 20
EMB_DIM = 64
AUG = EMB_DIM + 1
VOCAB = 100000

VT = 4352
NV = VPAD_ // VT
VPAD = VPAD_
VT2 = 2176
NV2 = VPAD // VT2
NEG = -1e30
LOG2E = 1.4426950408889634
LN2 = 0.6931471805599453

NUM_WORKERS = 32
N_IDX = BATCH * CONTEXT
B_PER_W = N_IDX // NUM_WORKERS
ROWS_PER_W = BATCH // NUM_WORKERS
LCHUNK = 16


def _idx_body(in_ref, o_ref):
    o_ref[...] = in_ref[...].T.reshape(N_IDX)


def _idx_flat(inputs):
    return pl.pallas_call(
        _idx_body,
        in_specs=[pl.BlockSpec((BATCH, CONTEXT), lambda: (0, 0))],
        out_specs=pl.BlockSpec((N_IDX,), lambda: (0,)),
        out_shape=jax.ShapeDtypeStruct((N_IDX,), jnp.int32),
    )(inputs)


TPT = 2048

def _tbl_body(in_ref, o_ref):
    o_ref[...] = jnp.pad(in_ref[...].T, ((0, 0), (0, EMB_DIM)))


def _tbl_repack(table_t):
    return pl.pallas_call(
        _tbl_body,
        grid=(-(-VOCAB // TPT),),
        in_specs=[pl.BlockSpec((EMB_DIM, TPT), lambda j: (0, j))],
        out_specs=pl.BlockSpec((TPT, 2 * EMB_DIM), lambda j: (j, 0)),
        out_shape=jax.ShapeDtypeStruct((VOCAB, 2 * EMB_DIM), jnp.float32),
    )(table_t)


def _sc_gather_sum_body(table_hbm, idx_hbm, out_hbm, idx_v, rows_v, acc_v, sem):
    wid = lax.axis_index("s") * 2 + lax.axis_index("c")
    col0 = wid * ROWS_PER_W
    for c in range(CONTEXT):
        pltpu.sync_copy(
            idx_hbm.at[pl.ds(c * BATCH + col0, ROWS_PER_W)],
            idx_v.at[pl.ds(c * ROWS_PER_W, ROWS_PER_W)],
        )
    pltpu.async_copy(table_hbm.at[idx_v], rows_v, sem).wait()

    @pl.loop(0, ROWS_PER_W)
    def _(r):
        for k in range(EMB_DIM // LCHUNK):
            sl = pl.ds(k * LCHUNK, LCHUNK)
            acc = rows_v[r, sl]
            for c in range(1, CONTEXT):
                acc = acc + rows_v[c * ROWS_PER_W + r, sl]
            acc_v[r, sl] = acc

    pltpu.sync_copy(acc_v, out_hbm.at[pl.ds(col0, ROWS_PER_W)])


def _sc_gather_sum(table128, idx):
    mesh = plsc.VectorSubcoreMesh(core_axis_name="c", subcore_axis_name="s")
    k = pl.kernel(
        _sc_gather_sum_body,
        out_type=jax.ShapeDtypeStruct((BATCH, 2 * EMB_DIM), jnp.float32),
        mesh=mesh,
        compiler_params=pltpu.CompilerParams(use_tc_tiling_on_sc=True),
        scratch_types=[
            pltpu.VMEM((B_PER_W,), jnp.int32),
            pltpu.VMEM((B_PER_W, 2 * EMB_DIM), jnp.float32),
            pltpu.VMEM((ROWS_PER_W, 2 * EMB_DIM), jnp.float32),
            pltpu.SemaphoreType.DMA,
        ],
    )
    return k(table128, idx)


def _stats_body(x_ref, w_ref, lse_ref, m_ref):
    j = pl.program_id(0)

    @pl.when(j == 0)
    def _init():
        m_ref[...] = jnp.full((1, BATCH), -jnp.inf, jnp.float32)
        lse_ref[...] = jnp.zeros((1, BATCH), jnp.float32)

    lt2 = lax.dot_general(
        w_ref[...], x_ref[...].astype(jnp.bfloat16), (((0,), (0,)), ((), ())),
        preferred_element_type=jnp.float32,
    )
    tmax = jnp.max(lt2, axis=0, keepdims=True)
    m_old = m_ref[...]
    m_new = jnp.maximum(m_old, tmax)
    e = jnp.exp2(lt2 - m_new)
    tsum = jnp.sum(e, axis=0, keepdims=True)
    s = lse_ref[...] * jnp.exp2(m_old - m_new) + tsum
    lse_ref[...] = s
    m_ref[...] = m_new

    @pl.when(j == NV - 1)
    def _fin():
        lse_ref[...] = (m_ref[...] + jnp.log2(lse_ref[...])) * LN2


def _stats(x_aug_t, w_aug):
    return pl.pallas_call(
        _stats_body,
        grid=(NV,),
        in_specs=[
            pl.BlockSpec((AUG, BATCH), lambda j: (0, 0)),
            pl.BlockSpec((AUG, VT), lambda j: (0, j)),
        ],
        out_specs=[
            pl.BlockSpec((1, BATCH), lambda j: (0, 0)),
            pl.BlockSpec((1, BATCH), lambda j: (0, 0)),
        ],
        out_shape=[
            jax.ShapeDtypeStruct((1, BATCH), jnp.float32),
            jax.ShapeDtypeStruct((1, BATCH), jnp.float32),
        ],
    )(x_aug_t, w_aug)


def _out_body(x_ref, w_ref, lse_ref, o_ref):
    lt = lax.dot_general(
        w_ref[...], x_ref[...].astype(jnp.bfloat16), (((0,), (0,)), ((), ())),
        preferred_element_type=jnp.float32,
    )
    o_ref[...] = lt * LN2 - lse_ref[...]


def _final(x_aug_t, w_aug, lse):
    return pl.pallas_call(
        _out_body,
        grid=(NV2,),
        in_specs=[
            pl.BlockSpec((AUG, BATCH), lambda j: (0, 0)),
            pl.BlockSpec((AUG, VT2), lambda j: (0, j)),
            pl.BlockSpec((1, BATCH), lambda j: (0, 0)),
        ],
        out_specs=pl.BlockSpec((VT2, BATCH), lambda j: (j, 0)),
        out_shape=jax.ShapeDtypeStruct((VOCAB, BATCH), jnp.float32),
    )(x_aug_t, w_aug, lse)


def kernel(inputs, emb_table, W, b):
    idx = _idx_flat(inputs.astype(jnp.int32))
    table128 = _tbl_repack(emb_table.T)
    xsum = _sc_gather_sum(table128, idx)

    x_aug_t = jnp.concatenate(
        [xsum[:, :EMB_DIM], jnp.ones((BATCH, 1), jnp.float32)], axis=1).T
    w_aug = (jnp.concatenate(
        [jnp.pad(W.T, ((0, 0), (0, VPAD - VOCAB))),
         jnp.pad(b, (0, VPAD - VOCAB), constant_values=NEG)[None, :]],
        axis=0) * LOG2E).astype(jnp.bfloat16)

    lse, _ = _stats(x_aug_t, w_aug)
    out_t = _final(x_aug_t, w_aug, lse)
    return out_t.T
